# baseline (speedup 1.0000x reference)
import functools

import jax
import jax.numpy as jnp
import numpy as np
from jax.experimental import pallas as pl
from jax.experimental.pallas import tpu as pltpu

_DIN = 512
_FEAT_STRIDE = 16
_A = 9
_PRE_NMS_TOPN = 6000
_POST_NMS_TOPN = 300
_NMS_THRESH = 0.7

_H, _W = 38, 63
_NPOS = _H * _W
_NANCH = _NPOS * _A
_NPAD = 22528
_ROWS = _NPAD // 128
_MPAD = 2432


def _whctrs_np(a):
    w = a[2] - a[0] + 1.0
    h = a[3] - a[1] + 1.0
    return w, h, a[0] + 0.5 * (w - 1.0), a[1] + 0.5 * (h - 1.0)


def _mkanchors_np(ws, hs, xc, yc):
    ws = ws[:, None]
    hs = hs[:, None]
    return np.hstack((xc - 0.5 * (ws - 1.0), yc - 0.5 * (hs - 1.0),
                      xc + 0.5 * (ws - 1.0), yc + 0.5 * (hs - 1.0)))


def _base_anchors_np():
    ratios = np.array([0.5, 1.0, 2.0])
    scales = np.array([8.0, 16.0, 32.0])
    base = np.array([0.0, 0.0, 15.0, 15.0])
    w, h, xc, yc = _whctrs_np(base)
    size = w * h
    sr = size / ratios
    ws = np.round(np.sqrt(sr))
    hs = np.round(ws * ratios)
    ra = _mkanchors_np(ws, hs, xc, yc)
    outs = []
    for i in range(ra.shape[0]):
        w2, h2, xc2, yc2 = _whctrs_np(ra[i])
        outs.append(_mkanchors_np(w2 * scales, h2 * scales, xc2, yc2))
    return np.vstack(outs).astype(np.float32)


def _anchor_planes_np():
    base = _base_anchors_np()
    sx = (np.arange(_W, dtype=np.float32) * _FEAT_STRIDE)
    sy = (np.arange(_H, dtype=np.float32) * _FEAT_STRIDE)
    mx, my = np.meshgrid(sx, sy)
    shifts = np.stack([mx.ravel(), my.ravel(), mx.ravel(), my.ravel()], axis=1)
    anch = (base[None, :, :] + shifts[:, None, :]).reshape(-1, 4).astype(np.float32)
    pad = np.zeros((_NPAD - _NANCH, 4), np.float32)
    anch = np.concatenate([anch, pad], axis=0)
    return [anch[:, i].reshape(_ROWS, 128) for i in range(4)]


_ANCH_PLANES = _anchor_planes_np()


def _head_kernel(x_ref, w_ref, b_ref, o_ref):
    x = jnp.maximum(x_ref[...], 0.0)
    o_ref[...] = jnp.dot(x, w_ref[...], preferred_element_type=jnp.float32) + b_ref[...]


def _run_head(conv1_raw_flat, w_cat, b_cat):
    B = conv1_raw_flat.shape[0]
    return pl.pallas_call(
        _head_kernel,
        grid=(B,),
        in_specs=[
            pl.BlockSpec((1, _MPAD, _DIN), lambda b: (b, 0, 0)),
            pl.BlockSpec((_DIN, 64), lambda b: (0, 0)),
            pl.BlockSpec((1, 64), lambda b: (0, 0)),
        ],
        out_specs=pl.BlockSpec((1, _MPAD, 64), lambda b: (b, 0, 0)),
        out_shape=jax.ShapeDtypeStruct((B, _MPAD, 64), jnp.float32),
    )(conv1_raw_flat, w_cat, b_cat)


def _nms_kernel(s0_ref, s1_ref, dx_ref, dy_ref, dw_ref, dh_ref,
                ax1_ref, ay1_ref, ax2_ref, ay2_ref, hi_ref, o_ref):
    shape = (_ROWS, 128)
    refk = (jax.lax.broadcasted_iota(jnp.int32, shape, 0) * 128
            + jax.lax.broadcasted_iota(jnp.int32, shape, 1))
    lane_valid = refk < _NANCH

    s0 = s0_ref[0]
    s1 = s1_ref[0]
    m = jnp.maximum(s0, s1)
    e0 = jnp.exp(s0 - m)
    e1 = jnp.exp(s1 - m)
    p = e1 / (e0 + e1)

    ax1 = ax1_ref[...]
    ay1 = ay1_ref[...]
    ax2 = ax2_ref[...]
    ay2 = ay2_ref[...]
    widths = ax2 - ax1 + 1.0
    heights = ay2 - ay1 + 1.0
    ctr_x = ax1 + 0.5 * widths
    ctr_y = ay1 + 0.5 * heights
    pcx = dx_ref[0] * widths + ctr_x
    pcy = dy_ref[0] * heights + ctr_y
    pw = jnp.exp(dw_ref[0]) * widths
    ph = jnp.exp(dh_ref[0]) * heights
    hi_x = hi_ref[0, 0, 0]
    hi_y = hi_ref[0, 0, 1]
    x1 = jnp.minimum(jnp.maximum(pcx - 0.5 * pw, 0.0), hi_x)
    y1 = jnp.minimum(jnp.maximum(pcy - 0.5 * ph, 0.0), hi_y)
    x2 = jnp.minimum(jnp.maximum(pcx + 0.5 * pw, 0.0), hi_x)
    y2 = jnp.minimum(jnp.maximum(pcy + 0.5 * ph, 0.0), hi_y)
    areas = (x2 - x1 + 1.0) * (y2 - y1 + 1.0)

    bits = jnp.where(lane_valid, jax.lax.bitcast_convert_type(p, jnp.int32),
                     jnp.int32(-1))

    def cnt_ge(t):
        return jnp.sum(jnp.where(bits >= t, jnp.int32(1), jnp.int32(0)))

    v = jnp.int32(0)
    for b in range(30, -1, -1):
        t = v | jnp.int32(1 << b)
        v = jnp.where(cnt_ge(t) >= _PRE_NMS_TOPN, t, v)
    n_above = jnp.sum(jnp.where(bits > v, jnp.int32(1), jnp.int32(0)))
    m_ties = _PRE_NMS_TOPN - n_above
    ties = bits == v

    def cnt_tie_below(kk):
        return jnp.sum(jnp.where(ties & (refk < kk), jnp.int32(1), jnp.int32(0)))

    g = jnp.int32(0)
    for b in range(15, -1, -1):
        t = g | jnp.int32(1 << b)
        g = jnp.where(cnt_tie_below(t) < m_ties, t, g)
    k_sel = jnp.where(m_ties > 0, g + 1, jnp.int32(0))

    cand = (bits > v) | (ties & (refk < k_sel))
    neg_inf = jnp.float32(-jnp.inf)
    s_init = jnp.where(cand, p, neg_inf)

    bi = hi_ref[0, 0, 2]
    lane = jax.lax.broadcasted_iota(jnp.int32, (1, 128), 1)
    big_k = jnp.int32(2 ** 30)
    nfold = _ROWS // 8

    def _allred(x, op):
        x = x.reshape(nfold, 8, 128)
        r = x[0]
        for i in range(1, nfold):
            r = op(r, x[i])
        for sh in (64, 32, 16, 8, 4, 2, 1):
            r = op(r, pltpu.roll(r, sh, 1))
        for sh in (4, 2, 1):
            r = op(r, pltpu.roll(r, sh, 0))
        return r

    def _bcast(r8):
        return jnp.broadcast_to(r8[None], (nfold, 8, 128)).reshape(_ROWS, 128)

    def body(t, s):
        m8 = _allred(s, jnp.maximum)
        mb = _bcast(m8)
        sel = s == mb
        km8 = _allred(jnp.where(sel, refk, big_k), jnp.minimum)
        oneh = refk == _bcast(km8)
        bx1 = _allred(jnp.where(oneh, x1, neg_inf), jnp.maximum)
        by1 = _allred(jnp.where(oneh, y1, neg_inf), jnp.maximum)
        bx2 = _allred(jnp.where(oneh, x2, neg_inf), jnp.maximum)
        by2 = _allred(jnp.where(oneh, y2, neg_inf), jnp.maximum)
        barea = _allred(jnp.where(oneh, areas, neg_inf), jnp.maximum)
        xx1 = jnp.maximum(_bcast(bx1), x1)
        yy1 = jnp.maximum(_bcast(by1), y1)
        xx2 = jnp.minimum(_bcast(bx2), x2)
        yy2 = jnp.minimum(_bcast(by2), y2)
        w = jnp.maximum(0.0, xx2 - xx1 + 1.0)
        h = jnp.maximum(0.0, yy2 - yy1 + 1.0)
        inter = w * h
        iou = inter / (_bcast(barea) + areas - inter)
        supp = (iou > _NMS_THRESH) | oneh
        s = jnp.where(supp, neg_inf, s)

        fv = jnp.where(m8[0:1, :] > neg_inf, 1.0, 0.0)
        row = jnp.where(lane == 0, bi,
              jnp.where(lane == 1, fv * bx1[0:1, :],
              jnp.where(lane == 2, fv * by1[0:1, :],
              jnp.where(lane == 3, fv * bx2[0:1, :],
              jnp.where(lane == 4, fv * by2[0:1, :], 0.0)))))
        o_ref[0, pl.ds(t, 1), :] = row
        return s

    jax.lax.fori_loop(0, _POST_NMS_TOPN, body, s_init)


def _run_nms(planes, anchors, hi):
    B = planes[0].shape[0]
    bspec = pl.BlockSpec((1, _ROWS, 128), lambda b: (b, 0, 0))
    aspec = pl.BlockSpec((_ROWS, 128), lambda b: (0, 0))
    return pl.pallas_call(
        _nms_kernel,
        grid=(B,),
        in_specs=[bspec] * 6 + [aspec] * 4 + [pl.BlockSpec((1, 8, 128), lambda b: (b, 0, 0))],
        out_specs=pl.BlockSpec((1, _POST_NMS_TOPN + 4, 128), lambda b: (b, 0, 0)),
        out_shape=jax.ShapeDtypeStruct((B, _POST_NMS_TOPN + 4, 128), jnp.float32),
    )(*planes, *anchors, hi)


def kernel(base_feat, im_info, gt_boxes, num_boxes, W_conv, b_conv,
           W_cls, b_cls, W_bbox, b_bbox):
    B = base_feat.shape[0]

    conv1 = jax.lax.conv_general_dilated(
        base_feat, W_conv, (1, 1), [(1, 1), (1, 1)],
        dimension_numbers=('NCHW', 'OIHW', 'NCHW'))
    conv1 = conv1 + b_conv[None, :, None, None]

    xflat = jnp.transpose(conv1, (0, 2, 3, 1)).reshape(B, _NPOS, _DIN)
    xflat = jnp.pad(xflat, ((0, 0), (0, _MPAD - _NPOS), (0, 0)))

    w_cat = jnp.concatenate([W_cls[:, :, 0, 0].T, W_bbox[:, :, 0, 0].T], axis=1)
    w_cat = jnp.pad(w_cat, ((0, 0), (0, 64 - 6 * _A)))
    b_cat = jnp.pad(jnp.concatenate([b_cls, b_bbox]), (0, 64 - 6 * _A))[None, :]

    head = _run_head(xflat, w_cat, b_cat)
    head = head[:, :_NPOS, :]

    def to_plane(x):
        flat = x.reshape(B, _NANCH)
        flat = jnp.pad(flat, ((0, 0), (0, _NPAD - _NANCH)))
        return flat.reshape(B, _ROWS, 128)

    s0 = to_plane(head[:, :, 0:_A])
    s1 = to_plane(head[:, :, _A:2 * _A])
    deltas = head[:, :, 2 * _A:6 * _A].reshape(B, _NPOS, _A, 4)
    dx = to_plane(deltas[..., 0])
    dy = to_plane(deltas[..., 1])
    dw = to_plane(deltas[..., 2])
    dh = to_plane(deltas[..., 3])

    anchors = [jnp.asarray(a) for a in _ANCH_PLANES]

    hi = jnp.zeros((B, 8, 128), jnp.float32)
    hi = hi.at[:, 0, 0].set(im_info[:, 1] - 1.0)
    hi = hi.at[:, 0, 1].set(im_info[:, 0] - 1.0)
    hi = hi.at[:, 0, 2].set(jnp.arange(B, dtype=jnp.float32))

    out = _run_nms([s0, s1, dx, dy, dw, dh], anchors, hi)
    rois = out[:, :_POST_NMS_TOPN, :5]

    zero = jnp.zeros((), dtype=jnp.float32)
    distil = jnp.zeros((1,), dtype=jnp.float32)
    return rois, zero, zero, distil


# speedup vs baseline: 2.4381x; 2.4381x over previous
import functools

import jax
import jax.numpy as jnp
import numpy as np
from jax.experimental import pallas as pl
from jax.experimental.pallas import tpu as pltpu

_DIN = 512
_FEAT_STRIDE = 16
_A = 9
_PRE_NMS_TOPN = 6000
_POST_NMS_TOPN = 300
_NMS_THRESH = 0.7

_H, _W = 38, 63
_NPOS = _H * _W
_NANCH = _NPOS * _A
_NPAD = 22528
_ROWS = _NPAD // 128
_MPAD = 2432


def _whctrs_np(a):
    w = a[2] - a[0] + 1.0
    h = a[3] - a[1] + 1.0
    return w, h, a[0] + 0.5 * (w - 1.0), a[1] + 0.5 * (h - 1.0)


def _mkanchors_np(ws, hs, xc, yc):
    ws = ws[:, None]
    hs = hs[:, None]
    return np.hstack((xc - 0.5 * (ws - 1.0), yc - 0.5 * (hs - 1.0),
                      xc + 0.5 * (ws - 1.0), yc + 0.5 * (hs - 1.0)))


def _base_anchors_np():
    ratios = np.array([0.5, 1.0, 2.0])
    scales = np.array([8.0, 16.0, 32.0])
    base = np.array([0.0, 0.0, 15.0, 15.0])
    w, h, xc, yc = _whctrs_np(base)
    size = w * h
    sr = size / ratios
    ws = np.round(np.sqrt(sr))
    hs = np.round(ws * ratios)
    ra = _mkanchors_np(ws, hs, xc, yc)
    outs = []
    for i in range(ra.shape[0]):
        w2, h2, xc2, yc2 = _whctrs_np(ra[i])
        outs.append(_mkanchors_np(w2 * scales, h2 * scales, xc2, yc2))
    return np.vstack(outs).astype(np.float32)


def _anchor_planes_np():
    base = _base_anchors_np()
    sx = (np.arange(_W, dtype=np.float32) * _FEAT_STRIDE)
    sy = (np.arange(_H, dtype=np.float32) * _FEAT_STRIDE)
    mx, my = np.meshgrid(sx, sy)
    shifts = np.stack([mx.ravel(), my.ravel(), mx.ravel(), my.ravel()], axis=1)
    anch = (base[None, :, :] + shifts[:, None, :]).reshape(-1, 4).astype(np.float32)
    pad = np.zeros((_NPAD - _NANCH, 4), np.float32)
    anch = np.concatenate([anch, pad], axis=0)
    return [anch[:, i].reshape(_ROWS, 128) for i in range(4)]


_ANCH_PLANES = _anchor_planes_np()


def _head_kernel(x_ref, w_ref, b_ref, o_ref):
    x = jnp.maximum(x_ref[...], 0.0)
    o_ref[...] = jnp.dot(x, w_ref[...], preferred_element_type=jnp.float32) + b_ref[...]


def _run_head(conv1_raw_flat, w_cat, b_cat):
    B = conv1_raw_flat.shape[0]
    return pl.pallas_call(
        _head_kernel,
        grid=(B,),
        in_specs=[
            pl.BlockSpec((1, _MPAD, _DIN), lambda b: (b, 0, 0)),
            pl.BlockSpec((_DIN, 64), lambda b: (0, 0)),
            pl.BlockSpec((1, 64), lambda b: (0, 0)),
        ],
        out_specs=pl.BlockSpec((1, _MPAD, 64), lambda b: (b, 0, 0)),
        out_shape=jax.ShapeDtypeStruct((B, _MPAD, 64), jnp.float32),
    )(conv1_raw_flat, w_cat, b_cat)


def _nms_kernel(s0_ref, s1_ref, dx_ref, dy_ref, dw_ref, dh_ref,
                ax1_ref, ay1_ref, ax2_ref, ay2_ref, hi_ref, o_ref):
    shape = (_ROWS, 128)
    refk = (jax.lax.broadcasted_iota(jnp.int32, shape, 0) * 128
            + jax.lax.broadcasted_iota(jnp.int32, shape, 1))
    lane_valid = refk < _NANCH
    neg_inf = jnp.float32(-jnp.inf)
    big_k = jnp.int32(2 ** 30)
    lane = jax.lax.broadcasted_iota(jnp.int32, (1, 128), 1)

    ax1 = ax1_ref[...]
    ay1 = ay1_ref[...]
    ax2 = ax2_ref[...]
    ay2 = ay2_ref[...]
    widths = ax2 - ax1 + 1.0
    heights = ay2 - ay1 + 1.0
    ctr_x = ax1 + 0.5 * widths
    ctr_y = ay1 + 0.5 * heights

    def decode_one(b):
        s0 = s0_ref[b]
        s1 = s1_ref[b]
        m = jnp.maximum(s0, s1)
        e0 = jnp.exp(s0 - m)
        e1 = jnp.exp(s1 - m)
        p = e1 / (e0 + e1)

        pcx = dx_ref[b] * widths + ctr_x
        pcy = dy_ref[b] * heights + ctr_y
        pw = jnp.exp(dw_ref[b]) * widths
        ph = jnp.exp(dh_ref[b]) * heights
        hi_x = hi_ref[b, 0, 0]
        hi_y = hi_ref[b, 0, 1]
        x1 = jnp.minimum(jnp.maximum(pcx - 0.5 * pw, 0.0), hi_x)
        y1 = jnp.minimum(jnp.maximum(pcy - 0.5 * ph, 0.0), hi_y)
        x2 = jnp.minimum(jnp.maximum(pcx + 0.5 * pw, 0.0), hi_x)
        y2 = jnp.minimum(jnp.maximum(pcy + 0.5 * ph, 0.0), hi_y)
        areas = (x2 - x1 + 1.0) * (y2 - y1 + 1.0)

        bits = jnp.where(lane_valid,
                         jax.lax.bitcast_convert_type(p, jnp.int32),
                         jnp.int32(-1))
        v = jnp.int32(0)
        for bb in range(30, -1, -1):
            t = v | jnp.int32(1 << bb)
            cnt = jnp.sum(jnp.where(bits >= t, jnp.int32(1), jnp.int32(0)))
            v = jnp.where(cnt >= _PRE_NMS_TOPN, t, v)
        n_above = jnp.sum(jnp.where(bits > v, jnp.int32(1), jnp.int32(0)))
        m_ties = _PRE_NMS_TOPN - n_above
        ties = bits == v
        g = jnp.int32(0)
        for bb in range(15, -1, -1):
            t = g | jnp.int32(1 << bb)
            cnt = jnp.sum(jnp.where(ties & (refk < t), jnp.int32(1), jnp.int32(0)))
            g = jnp.where(cnt < m_ties, t, g)
        k_sel = jnp.where(m_ties > 0, g + 1, jnp.int32(0))
        cand = (bits > v) | (ties & (refk < k_sel))
        s_init = jnp.where(cand, p, neg_inf)
        return s_init, x1, y1, x2, y2, areas

    st0 = decode_one(0)
    st1 = decode_one(1)
    bi0 = hi_ref[0, 0, 2]
    bi1 = hi_ref[1, 0, 2]

    def pick(s, x1, y1, x2, y2, areas, bi, t, bslot):
        mx = jnp.max(s)
        valid = mx > neg_inf
        sel = s == mx
        km = jnp.min(jnp.where(sel, refk, big_k))
        oneh = refk == km
        bx1 = jnp.sum(jnp.where(oneh, x1, 0.0))
        by1 = jnp.sum(jnp.where(oneh, y1, 0.0))
        bx2 = jnp.sum(jnp.where(oneh, x2, 0.0))
        by2 = jnp.sum(jnp.where(oneh, y2, 0.0))
        barea = jnp.sum(jnp.where(oneh, areas, 0.0))
        xx1 = jnp.maximum(bx1, x1)
        yy1 = jnp.maximum(by1, y1)
        xx2 = jnp.minimum(bx2, x2)
        yy2 = jnp.minimum(by2, y2)
        w = jnp.maximum(0.0, xx2 - xx1 + 1.0)
        h = jnp.maximum(0.0, yy2 - yy1 + 1.0)
        inter = w * h
        iou = inter / (barea + areas - inter)
        supp = (iou > _NMS_THRESH) | oneh
        s = jnp.where(supp, neg_inf, s)
        fv = jnp.float32(valid)
        row = jnp.where(lane == 0, bi,
              jnp.where(lane == 1, fv * bx1,
              jnp.where(lane == 2, fv * by1,
              jnp.where(lane == 3, fv * bx2,
              jnp.where(lane == 4, fv * by2, 0.0)))))
        o_ref[bslot, pl.ds(t, 1), :] = row
        return s

    def body(t, carry):
        sa, sb = carry
        sa = pick(sa, st0[1], st0[2], st0[3], st0[4], st0[5], bi0, t, 0)
        sb = pick(sb, st1[1], st1[2], st1[3], st1[4], st1[5], bi1, t, 1)
        return sa, sb

    jax.lax.fori_loop(0, _POST_NMS_TOPN, body, (st0[0], st1[0]))


def _run_nms(planes, anchors, hi):
    bspec = pl.BlockSpec((2, _ROWS, 128), lambda: (0, 0, 0))
    aspec = pl.BlockSpec((_ROWS, 128), lambda: (0, 0))
    return pl.pallas_call(
        _nms_kernel,
        grid=(),
        in_specs=[bspec] * 6 + [aspec] * 4 + [pl.BlockSpec((2, 8, 128), lambda: (0, 0, 0))],
        out_specs=pl.BlockSpec((2, _POST_NMS_TOPN + 4, 128), lambda: (0, 0, 0)),
        out_shape=jax.ShapeDtypeStruct((2, _POST_NMS_TOPN + 4, 128), jnp.float32),
    )(*planes, *anchors, hi)


def kernel(base_feat, im_info, gt_boxes, num_boxes, W_conv, b_conv,
           W_cls, b_cls, W_bbox, b_bbox):
    B = base_feat.shape[0]

    conv1 = jax.lax.conv_general_dilated(
        base_feat, W_conv, (1, 1), [(1, 1), (1, 1)],
        dimension_numbers=('NCHW', 'OIHW', 'NCHW'))
    conv1 = conv1 + b_conv[None, :, None, None]

    xflat = jnp.transpose(conv1, (0, 2, 3, 1)).reshape(B, _NPOS, _DIN)
    xflat = jnp.pad(xflat, ((0, 0), (0, _MPAD - _NPOS), (0, 0)))

    w_cat = jnp.concatenate([W_cls[:, :, 0, 0].T, W_bbox[:, :, 0, 0].T], axis=1)
    w_cat = jnp.pad(w_cat, ((0, 0), (0, 64 - 6 * _A)))
    b_cat = jnp.pad(jnp.concatenate([b_cls, b_bbox]), (0, 64 - 6 * _A))[None, :]

    head = _run_head(xflat, w_cat, b_cat)
    head = head[:, :_NPOS, :]

    def to_plane(x):
        flat = x.reshape(B, _NANCH)
        flat = jnp.pad(flat, ((0, 0), (0, _NPAD - _NANCH)))
        return flat.reshape(B, _ROWS, 128)

    s0 = to_plane(head[:, :, 0:_A])
    s1 = to_plane(head[:, :, _A:2 * _A])
    deltas = head[:, :, 2 * _A:6 * _A].reshape(B, _NPOS, _A, 4)
    dx = to_plane(deltas[..., 0])
    dy = to_plane(deltas[..., 1])
    dw = to_plane(deltas[..., 2])
    dh = to_plane(deltas[..., 3])

    anchors = [jnp.asarray(a) for a in _ANCH_PLANES]

    hi = jnp.zeros((B, 8, 128), jnp.float32)
    hi = hi.at[:, 0, 0].set(im_info[:, 1] - 1.0)
    hi = hi.at[:, 0, 1].set(im_info[:, 0] - 1.0)
    hi = hi.at[:, 0, 2].set(jnp.arange(B, dtype=jnp.float32))

    out = _run_nms([s0, s1, dx, dy, dw, dh], anchors, hi)
    rois = out[:, :_POST_NMS_TOPN, :5]

    zero = jnp.zeros((), dtype=jnp.float32)
    distil = jnp.zeros((1,), dtype=jnp.float32)
    return rois, zero, zero, distil
